# baseline (speedup 1.0000x reference)
import functools

import jax
import jax.numpy as jnp
from jax import lax
from jax.experimental import pallas as pl
from jax.experimental.pallas import tpu as pltpu
from jax.experimental.pallas import tpu_sc as plsc

N_NODES = 50000
N_REL = 8
N_EDGES = 1600000
EMB = 16

NC = 2
NS = 16
LANES = 16

CHUNK = 1600
UNROLL = 8
E_PAD = 1638400
EDGES_PER_TILE = E_PAD // (NC * NS)
N_CHUNKS = EDGES_PER_TILE // CHUNK
N_GROUPS = N_CHUNKS // UNROLL

HP = 3128
NPAD = NS * HP
DP = 3128
DPAD = NS * DP


def _sc_accumulate(w2d, src, dst, rel):
    mesh = plsc.VectorSubcoreMesh(core_axis_name="c", subcore_axis_name="s")

    @functools.partial(
        pl.kernel,
        mesh=mesh,
        compiler_params=pltpu.CompilerParams(use_tc_tiling_on_sc=False),
        out_type=[
            jax.ShapeDtypeStruct((NC * NPAD, EMB), jnp.float32),
            jax.ShapeDtypeStruct((NC * DPAD,), jnp.float32),
        ],
        scratch_types=[
            pltpu.VMEM((CHUNK,), jnp.int32),
            pltpu.VMEM((CHUNK,), jnp.int32),
            pltpu.VMEM((CHUNK,), jnp.int32),
            pltpu.VMEM((CHUNK,), jnp.int32),
            pltpu.VMEM((CHUNK,), jnp.int32),
            pltpu.VMEM((CHUNK,), jnp.int32),
            pltpu.VMEM((CHUNK, EMB), jnp.float32),
            pltpu.VMEM((CHUNK, EMB), jnp.float32),
            pltpu.VMEM((CHUNK,), jnp.float32),
            pltpu.VMEM((CHUNK,), jnp.float32),
            pltpu.VMEM_SHARED((NPAD, EMB), jnp.float32),
            pltpu.VMEM_SHARED((DPAD,), jnp.float32),
            pltpu.SemaphoreType.DMA,
            pltpu.SemaphoreType.DMA,
            pltpu.SemaphoreType.DMA,
            pltpu.SemaphoreType.DMA,
            pltpu.SemaphoreType.DMA,
        ],
    )
    def k(w_hbm, src_hbm, dst_hbm, rel_hbm, h_out, d_out,
          src0, src1, src2, dstc0, dstc1, rel_v, rows0, rows1,
          ones_v, zero_v, h_sh, d_sh,
          sem_in, sem_g0, sem_g1, sem_sh, sem_sd):
        c = lax.axis_index("c")
        s = lax.axis_index("s")
        tile = c * NS + s
        srcs = [src0, src1, src2]
        dstc = [dstc0, dstc1]
        rows = [rows0, rows1]
        sem_g = [sem_g0, sem_g1]

        zeros16 = jnp.zeros((LANES,), jnp.float32)
        ones16 = jnp.ones((LANES,), jnp.float32)

        def init_body(i, _):
            rows0[i, :] = zeros16
            return _
        lax.fori_loop(0, CHUNK, init_body, None)

        def init_flat(i, _):
            ones_v[pl.ds(i * LANES, LANES)] = ones16
            zero_v[pl.ds(i * LANES, LANES)] = zeros16
            return _
        lax.fori_loop(0, CHUNK // LANES, init_flat, None)

        hbase = s * HP
        pltpu.sync_copy(rows0, h_sh.at[pl.ds(hbase, CHUNK)])
        pltpu.sync_copy(rows0.at[pl.ds(0, HP - CHUNK)],
                        h_sh.at[pl.ds(hbase + CHUNK, HP - CHUNK)])
        dbase = s * DP
        pltpu.sync_copy(zero_v, d_sh.at[pl.ds(dbase, CHUNK)])
        pltpu.sync_copy(zero_v.at[pl.ds(0, DP - CHUNK)],
                        d_sh.at[pl.ds(dbase + CHUNK, DP - CHUNK)])

        plsc.subcore_barrier()

        ebase = tile * EDGES_PER_TILE

        def load_inputs(goff, j):
            off = goff + j * CHUNK
            pltpu.sync_copy(src_hbm.at[pl.ds(off, CHUNK)], srcs[j % 3])
            pltpu.sync_copy(dst_hbm.at[pl.ds(off, CHUNK)], dstc[j % 2])
            pltpu.sync_copy(rel_hbm.at[pl.ds(off, CHUNK)], rel_v)

        def compute_col(j):
            b = j % 2

            def col_body(i, _):
                sl = pl.ds(i * LANES, LANES)
                dstc[b][sl] = rel_v[sl] * jnp.int32(N_NODES) + dstc[b][sl]
                return _
            lax.fori_loop(0, CHUNK // LANES, col_body, None)

        def scatter_pair(j):
            return [
                pltpu.async_copy(rows[j % 2], h_sh.at[srcs[j % 3]], sem_sh,
                                 add=True),
                pltpu.async_copy(ones_v, d_sh.at[srcs[j % 3]], sem_sd,
                                 add=True),
            ]

        def group_body(g, _):
            goff = ebase + g * (UNROLL * CHUNK)
            g_h = {}
            s_h = {}
            for j in range(UNROLL):
                load_inputs(goff, j)
                compute_col(j)
                if j >= 2:
                    for h in s_h.pop(j - 2):
                        h.wait()
                g_h[j] = pltpu.async_copy(
                    w_hbm.at[dstc[j % 2]], rows[j % 2], sem_g[j % 2])
                if j >= 1:
                    g_h.pop(j - 1).wait()
                    s_h[j - 1] = scatter_pair(j - 1)
            for h in s_h.pop(UNROLL - 2):
                h.wait()
            g_h.pop(UNROLL - 1).wait()
            for h in scatter_pair(UNROLL - 1):
                h.wait()
            return _
        lax.fori_loop(0, N_GROUPS, group_body, None)

        plsc.subcore_barrier()

        pltpu.sync_copy(h_sh.at[pl.ds(hbase, HP)],
                        h_out.at[pl.ds(c * NPAD + hbase, HP)])
        pltpu.sync_copy(d_sh.at[pl.ds(dbase, DP)],
                        d_out.at[pl.ds(c * DPAD + dbase, DP)])

    return k(w2d, src, dst, rel)


def _combine(p0, p1, d0, d1):
    def body(p0_ref, p1_ref, d0_ref, d1_ref, o_ref):
        deg = d0_ref[...] + d1_ref[...]
        scale = 1.0 / jnp.maximum(deg, 1.0)
        o_ref[...] = (p0_ref[...] + p1_ref[...]) * scale

    BR = 5000
    return pl.pallas_call(
        body,
        grid=(N_NODES // BR,),
        in_specs=[
            pl.BlockSpec((BR, EMB), lambda i: (i, 0)),
            pl.BlockSpec((BR, EMB), lambda i: (i, 0)),
            pl.BlockSpec((BR, 1), lambda i: (i, 0)),
            pl.BlockSpec((BR, 1), lambda i: (i, 0)),
        ],
        out_specs=pl.BlockSpec((BR, EMB), lambda i: (i, 0)),
        out_shape=jax.ShapeDtypeStruct((N_NODES, EMB), jnp.float32),
    )(p0, p1, d0, d1)


def kernel(weights, edge_src, edge_dst, edge_rel):
    w2d = weights.reshape(N_REL * N_NODES, EMB)
    src = edge_src.astype(jnp.int32)
    dst = edge_dst.astype(jnp.int32)
    rel = edge_rel.astype(jnp.int32)
    pad = E_PAD - N_EDGES
    pad_src = N_NODES + (jnp.arange(pad, dtype=jnp.int32) % 48)
    zpad = jnp.zeros((pad,), jnp.int32)
    src = jnp.concatenate([src, pad_src])
    dst = jnp.concatenate([dst, zpad])
    rel = jnp.concatenate([rel, zpad])
    h_part, d_part = _sc_accumulate(w2d, src, dst, rel)
    p0 = h_part[:N_NODES]
    p1 = h_part[NPAD:NPAD + N_NODES]
    d0 = d_part[:N_NODES].reshape(N_NODES, 1)
    d1 = d_part[DPAD:DPAD + N_NODES].reshape(N_NODES, 1)
    return _combine(p0, p1, d0, d1)


# speedup vs baseline: 1.3165x; 1.3165x over previous
import functools

import jax
import jax.numpy as jnp
from jax import lax
from jax.experimental import pallas as pl
from jax.experimental.pallas import tpu as pltpu
from jax.experimental.pallas import tpu_sc as plsc

N_NODES = 50000
N_REL = 8
N_EDGES = 1600000
EMB = 16

NC = 2
NS = 16
LANES = 16

CHUNK = 2000
UNROLL = 5
EDGES_PER_TILE = N_EDGES // (NC * NS)
N_CHUNKS = EDGES_PER_TILE // CHUNK
N_GROUPS = N_CHUNKS // UNROLL

HP = 3128
NPAD = NS * HP
DP = 3128
DPAD = NS * DP


def _sc_accumulate(w2d, src, dst, rel, zeros_d):
    mesh = plsc.VectorSubcoreMesh(core_axis_name="c", subcore_axis_name="s")

    @functools.partial(
        pl.kernel,
        mesh=mesh,
        compiler_params=pltpu.CompilerParams(use_tc_tiling_on_sc=False),
        out_type=[
            jax.ShapeDtypeStruct((NC * NPAD, EMB), jnp.float32),
            jax.ShapeDtypeStruct((NC * DPAD,), jnp.float32),
        ],
        scratch_types=[
            pltpu.VMEM((CHUNK,), jnp.int32),
            pltpu.VMEM((CHUNK,), jnp.int32),
            pltpu.VMEM((CHUNK,), jnp.int32),
            pltpu.VMEM((CHUNK,), jnp.int32),
            pltpu.VMEM((CHUNK,), jnp.int32),
            pltpu.VMEM((CHUNK, EMB), jnp.float32),
            pltpu.VMEM((CHUNK, EMB), jnp.float32),
            pltpu.VMEM((CHUNK,), jnp.float32),
            pltpu.VMEM_SHARED((NPAD, EMB), jnp.float32),
            pltpu.VMEM_SHARED((DPAD,), jnp.float32),
            pltpu.SemaphoreType.DMA,
            pltpu.SemaphoreType.DMA,
            pltpu.SemaphoreType.DMA,
            pltpu.SemaphoreType.DMA,
            pltpu.SemaphoreType.DMA,
        ],
    )
    def k(w_hbm, src_hbm, dst_hbm, rel_hbm, zd_hbm, h_out, d_out,
          src0, src1, dstc0, dstc1, rel_v, rows0, rows1, ones_v,
          h_sh, d_sh, sem_in, sem_g0, sem_g1, sem_sh, sem_sd):
        c = lax.axis_index("c")
        s = lax.axis_index("s")
        tile = c * NS + s
        srcs = [src0, src1]
        dstc = [dstc0, dstc1]
        rows = [rows0, rows1]
        sem_g = [sem_g0, sem_g1]

        zeros16 = jnp.zeros((LANES,), jnp.float32)
        ones16 = jnp.ones((LANES,), jnp.float32)

        def init_body(i, _):
            rows0[i, :] = zeros16
            return _
        lax.fori_loop(0, CHUNK, init_body, None)

        def init_flat(i, _):
            ones_v[pl.ds(i * LANES, LANES)] = ones16
            return _
        lax.fori_loop(0, CHUNK // LANES, init_flat, None)

        hbase = s * HP
        pltpu.sync_copy(rows0, h_sh.at[pl.ds(hbase, CHUNK)])
        pltpu.sync_copy(rows0.at[pl.ds(0, HP - CHUNK)],
                        h_sh.at[pl.ds(hbase + CHUNK, HP - CHUNK)])
        dbase = s * DP
        pltpu.sync_copy(zd_hbm, d_sh.at[pl.ds(dbase, DP)])

        plsc.subcore_barrier()

        ebase = tile * EDGES_PER_TILE

        def load_inputs(goff, j):
            off = goff + j * CHUNK
            pltpu.sync_copy(src_hbm.at[pl.ds(off, CHUNK)], srcs[j % 2])
            pltpu.sync_copy(dst_hbm.at[pl.ds(off, CHUNK)], dstc[j % 2])
            pltpu.sync_copy(rel_hbm.at[pl.ds(off, CHUNK)], rel_v)

        def compute_col(j):
            b = j % 2

            def col_body(i, _):
                sl = pl.ds(i * LANES, LANES)
                dstc[b][sl] = rel_v[sl] * jnp.int32(N_NODES) + dstc[b][sl]
                return _
            lax.fori_loop(0, CHUNK // LANES, col_body, None)

        def issue_gather(j):
            return pltpu.async_copy(
                w_hbm.at[dstc[j % 2]], rows[j % 2], sem_g[j % 2])

        def issue_scatters(j):
            return [
                pltpu.async_copy(rows[j % 2], h_sh.at[srcs[j % 2]], sem_sh,
                                 add=True),
                pltpu.async_copy(ones_v, d_sh.at[srcs[j % 2]], sem_sd,
                                 add=True),
            ]

        def group_body(g, _):
            goff = ebase + g * (UNROLL * CHUNK)
            load_inputs(goff, 0)
            compute_col(0)
            g_h = issue_gather(0)
            s_h = None
            for j in range(UNROLL):
                if s_h is not None:
                    for h in s_h:
                        h.wait()
                g_h.wait()
                s_h = issue_scatters(j)
                if j + 1 < UNROLL:
                    load_inputs(goff, j + 1)
                    compute_col(j + 1)
                    g_h = issue_gather(j + 1)
            for h in s_h:
                h.wait()
            return _
        lax.fori_loop(0, N_GROUPS, group_body, None)

        plsc.subcore_barrier()

        pltpu.sync_copy(h_sh.at[pl.ds(hbase, HP)],
                        h_out.at[pl.ds(c * NPAD + hbase, HP)])
        pltpu.sync_copy(d_sh.at[pl.ds(dbase, DP)],
                        d_out.at[pl.ds(c * DPAD + dbase, DP)])

    return k(w2d, src, dst, rel, zeros_d)


def _combine(p0, p1, d0, d1):
    def body(p0_ref, p1_ref, d0_ref, d1_ref, o_ref):
        deg = d0_ref[...] + d1_ref[...]
        scale = 1.0 / jnp.maximum(deg, 1.0)
        o_ref[...] = (p0_ref[...] + p1_ref[...]) * scale

    BR = 5000
    return pl.pallas_call(
        body,
        grid=(N_NODES // BR,),
        in_specs=[
            pl.BlockSpec((BR, EMB), lambda i: (i, 0)),
            pl.BlockSpec((BR, EMB), lambda i: (i, 0)),
            pl.BlockSpec((BR, 1), lambda i: (i, 0)),
            pl.BlockSpec((BR, 1), lambda i: (i, 0)),
        ],
        out_specs=pl.BlockSpec((BR, EMB), lambda i: (i, 0)),
        out_shape=jax.ShapeDtypeStruct((N_NODES, EMB), jnp.float32),
    )(p0, p1, d0, d1)


def kernel(weights, edge_src, edge_dst, edge_rel):
    w2d = weights.reshape(N_REL * N_NODES, EMB)
    src = edge_src.astype(jnp.int32)
    dst = edge_dst.astype(jnp.int32)
    rel = edge_rel.astype(jnp.int32)
    zeros_d = jnp.zeros((DP,), jnp.float32)
    h_part, d_part = _sc_accumulate(w2d, src, dst, rel, zeros_d)
    p0 = h_part[:N_NODES]
    p1 = h_part[NPAD:NPAD + N_NODES]
    d0 = d_part[:N_NODES].reshape(N_NODES, 1)
    d1 = d_part[DPAD:DPAD + N_NODES].reshape(N_NODES, 1)
    return _combine(p0, p1, d0, d1)


# speedup vs baseline: 1.7276x; 1.3123x over previous
import functools

import jax
import jax.numpy as jnp
from jax import lax
from jax.experimental import pallas as pl
from jax.experimental.pallas import tpu as pltpu
from jax.experimental.pallas import tpu_sc as plsc

N_NODES = 50000
N_REL = 8
N_EDGES = 1600000
EMB = 16

NC = 2
NS = 16
LANES = 16

CHUNK = 2000
UNROLL = 25
EDGES_PER_TILE = N_EDGES // (NC * NS)
N_CHUNKS = EDGES_PER_TILE // CHUNK
N_GROUPS = N_CHUNKS // UNROLL

HP = 3128
NPAD = NS * HP
DP = 3128
DPAD = NS * DP


def _sc_accumulate(w2d, src, dst, rel, zeros_d):
    mesh = plsc.VectorSubcoreMesh(core_axis_name="c", subcore_axis_name="s")

    @functools.partial(
        pl.kernel,
        mesh=mesh,
        compiler_params=pltpu.CompilerParams(use_tc_tiling_on_sc=False),
        out_type=[
            jax.ShapeDtypeStruct((NC * NPAD, EMB), jnp.float32),
            jax.ShapeDtypeStruct((NC * DPAD,), jnp.float32),
        ],
        scratch_types=[
            pltpu.VMEM((CHUNK,), jnp.int32),
            pltpu.VMEM((CHUNK,), jnp.int32),
            pltpu.VMEM((CHUNK,), jnp.int32),
            pltpu.VMEM((CHUNK,), jnp.int32),
            pltpu.VMEM((CHUNK,), jnp.int32),
            pltpu.VMEM((CHUNK, EMB), jnp.float32),
            pltpu.VMEM((CHUNK, EMB), jnp.float32),
            pltpu.VMEM((CHUNK,), jnp.float32),
            pltpu.VMEM_SHARED((NPAD, EMB), jnp.float32),
            pltpu.VMEM_SHARED((DPAD,), jnp.float32),
            pltpu.SemaphoreType.DMA,
            pltpu.SemaphoreType.DMA,
            pltpu.SemaphoreType.DMA,
            pltpu.SemaphoreType.DMA,
            pltpu.SemaphoreType.DMA,
        ],
    )
    def k(w_hbm, src_hbm, dst_hbm, rel_hbm, zd_hbm, h_out, d_out,
          src0, src1, dstc0, dstc1, rel_v, rows0, rows1, ones_v,
          h_sh, d_sh, sem_in, sem_g0, sem_g1, sem_sh, sem_sd):
        c = lax.axis_index("c")
        s = lax.axis_index("s")
        tile = c * NS + s
        srcs = [src0, src1]
        dstc = [dstc0, dstc1]
        rows = [rows0, rows1]
        sem_g = [sem_g0, sem_g1]

        zeros16 = jnp.zeros((LANES,), jnp.float32)
        ones16 = jnp.ones((LANES,), jnp.float32)

        def init_body(i, _):
            rows0[i, :] = zeros16
            return _
        lax.fori_loop(0, CHUNK, init_body, None)

        def init_flat(i, _):
            ones_v[pl.ds(i * LANES, LANES)] = ones16
            return _
        lax.fori_loop(0, CHUNK // LANES, init_flat, None)

        hbase = s * HP
        pltpu.sync_copy(rows0, h_sh.at[pl.ds(hbase, CHUNK)])
        pltpu.sync_copy(rows0.at[pl.ds(0, HP - CHUNK)],
                        h_sh.at[pl.ds(hbase + CHUNK, HP - CHUNK)])
        dbase = s * DP
        pltpu.sync_copy(zd_hbm, d_sh.at[pl.ds(dbase, DP)])

        plsc.subcore_barrier()

        ebase = tile * EDGES_PER_TILE

        def load_inputs(goff, j):
            off = goff + j * CHUNK
            pltpu.sync_copy(src_hbm.at[pl.ds(off, CHUNK)], srcs[j % 2])
            pltpu.sync_copy(dst_hbm.at[pl.ds(off, CHUNK)], dstc[j % 2])
            pltpu.sync_copy(rel_hbm.at[pl.ds(off, CHUNK)], rel_v)

        def compute_col(j):
            b = j % 2

            def col_body(i, _):
                sl = pl.ds(i * LANES, LANES)
                dstc[b][sl] = rel_v[sl] * jnp.int32(N_NODES) + dstc[b][sl]
                return _
            lax.fori_loop(0, CHUNK // LANES, col_body, None)

        def issue_gather(j):
            return pltpu.async_copy(
                w_hbm.at[dstc[j % 2]], rows[j % 2], sem_g[j % 2])

        def issue_scatters(j):
            return [
                pltpu.async_copy(rows[j % 2], h_sh.at[srcs[j % 2]], sem_sh,
                                 add=True),
                pltpu.async_copy(ones_v, d_sh.at[srcs[j % 2]], sem_sd,
                                 add=True),
            ]

        def group_body(g, _):
            goff = ebase + g * (UNROLL * CHUNK)
            load_inputs(goff, 0)
            compute_col(0)
            g_h = issue_gather(0)
            s_h = None
            for j in range(UNROLL):
                if s_h is not None:
                    for h in s_h:
                        h.wait()
                g_h.wait()
                s_h = issue_scatters(j)
                if j + 1 < UNROLL:
                    load_inputs(goff, j + 1)
                    compute_col(j + 1)
                    g_h = issue_gather(j + 1)
            for h in s_h:
                h.wait()
            return _
        lax.fori_loop(0, N_GROUPS, group_body, None)

        plsc.subcore_barrier()

        pltpu.sync_copy(h_sh.at[pl.ds(hbase, HP)],
                        h_out.at[pl.ds(c * NPAD + hbase, HP)])
        pltpu.sync_copy(d_sh.at[pl.ds(dbase, DP)],
                        d_out.at[pl.ds(c * DPAD + dbase, DP)])

    return k(w2d, src, dst, rel, zeros_d)


CQ = 1568
CQ_LAST = N_NODES - 31 * CQ


def _sc_combine(h_part, d_part):
    mesh = plsc.VectorSubcoreMesh(core_axis_name="c", subcore_axis_name="s")

    @functools.partial(
        pl.kernel,
        mesh=mesh,
        compiler_params=pltpu.CompilerParams(use_tc_tiling_on_sc=False),
        out_type=[jax.ShapeDtypeStruct((N_NODES, EMB), jnp.float32)],
        scratch_types=[
            pltpu.VMEM((CQ, EMB), jnp.float32),
            pltpu.VMEM((CQ, EMB), jnp.float32),
            pltpu.VMEM((CQ,), jnp.float32),
            pltpu.VMEM((CQ,), jnp.float32),
        ],
    )
    def k(hp_hbm, dp_hbm, out, h0_v, h1_v, d0_v, d1_v):
        c = lax.axis_index("c")
        s = lax.axis_index("s")
        tile = c * NS + s
        base = tile * CQ
        ones16 = jnp.ones((LANES,), jnp.float32)

        def do(rows):
            pltpu.sync_copy(hp_hbm.at[pl.ds(base, rows)],
                            h0_v.at[pl.ds(0, rows)])
            pltpu.sync_copy(hp_hbm.at[pl.ds(NPAD + base, rows)],
                            h1_v.at[pl.ds(0, rows)])
            pltpu.sync_copy(dp_hbm.at[pl.ds(base, rows)],
                            d0_v.at[pl.ds(0, rows)])
            pltpu.sync_copy(dp_hbm.at[pl.ds(DPAD + base, rows)],
                            d1_v.at[pl.ds(0, rows)])

            def scale_body(i, _):
                sl = pl.ds(i * LANES, LANES)
                deg = d0_v[sl] + d1_v[sl]
                d0_v[sl] = ones16 / jnp.maximum(deg, 1.0)
                return _
            lax.fori_loop(0, rows // LANES, scale_body, None)

            def row_body(g, _):
                s_vec = d0_v[pl.ds(g * LANES, LANES)]
                for j in range(LANES):
                    i = g * LANES + j
                    h0_v[i, :] = (h0_v[i, :] + h1_v[i, :]) * s_vec[j]
                return _
            lax.fori_loop(0, rows // LANES, row_body, None)

            pltpu.sync_copy(h0_v.at[pl.ds(0, rows)],
                            out.at[pl.ds(base, rows)])

        @pl.when(tile < 31)
        def _():
            do(CQ)

        @pl.when(tile == 31)
        def _():
            do(CQ_LAST)

    out, = k(h_part, d_part)
    return out


def kernel(weights, edge_src, edge_dst, edge_rel):
    w2d = weights.reshape(N_REL * N_NODES, EMB)
    src = edge_src.astype(jnp.int32)
    dst = edge_dst.astype(jnp.int32)
    rel = edge_rel.astype(jnp.int32)
    zeros_d = jnp.zeros((DP,), jnp.float32)
    h_part, d_part = _sc_accumulate(w2d, src, dst, rel, zeros_d)
    return _sc_combine(h_part, d_part)
